# baseline (speedup 1.0000x reference)
import functools
import math

import jax
import jax.numpy as jnp
from jax import lax
from jax.experimental import pallas as pl
from jax.experimental.pallas import tpu as pltpu
from jax.experimental.pallas import tpu_sc as plsc

_SQRT2 = math.sqrt(2.0)


def _gelu(v):
    return 0.5 * v * (1.0 + lax.erf(v / _SQRT2))


def _tc1_body(nf, ew, w0, b0, w1, b1, wp0, bp0, wp1, bp1, x_out, prep_out, s_out):
    x = _gelu(jnp.dot(nf[...], w0[...], preferred_element_type=jnp.float32) + b0[...])
    x = _gelu(jnp.dot(x, w1[...], preferred_element_type=jnp.float32) + b1[...])
    x_out[...] = x
    p = _gelu(jnp.dot(x, wp0[...], preferred_element_type=jnp.float32) + bp0[...])
    p = _gelu(jnp.dot(p, wp1[...], preferred_element_type=jnp.float32) + bp1[...])
    prep_out[...] = p
    s_out[...] = jnp.sum(ew[...], keepdims=True).reshape(1, 1)


_SC_CORES = 2
_SC_SUBCORES = 16
_SUB = 128
_CHUNK = 512
_NSUB = _CHUNK // _SUB
_NCH = 20


def _sc_edge_kernel(n_nodes, n_edges_pad, h):
    n_cores, n_sub = _SC_CORES, _SC_SUBCORES
    mesh = plsc.VectorSubcoreMesh(core_axis_name="c", subcore_axis_name="s",
                                  num_cores=n_cores, num_subcores=n_sub)
    n_workers = n_cores * n_sub
    assert n_edges_pad == n_workers * _NCH * _CHUNK
    rpt = (-(-n_nodes // n_sub) + 7) // 8 * 8
    rpt_last = n_nodes - rpt * (n_sub - 1)
    assert rpt % 8 == 0 and rpt_last > 0

    @functools.partial(
        pl.kernel,
        mesh=mesh,
        out_type=[
            jax.ShapeDtypeStruct((n_nodes, h), jnp.float32),
            jax.ShapeDtypeStruct((n_nodes, h), jnp.float32),
        ],
        scratch_types=[
            pltpu.VMEM((3, _NSUB, _SUB), jnp.int32),
            pltpu.VMEM((3, _NSUB, _SUB), jnp.int32),
            pltpu.VMEM((3, _CHUNK), jnp.float32),
            pltpu.VMEM((2, _CHUNK, h), jnp.float32),
            pltpu.VMEM_SHARED((n_nodes, h), jnp.float32),
            pltpu.SemaphoreType.DMA,
            pltpu.SemaphoreType.DMA,
            pltpu.SemaphoreType.DMA,
            pltpu.SemaphoreType.DMA,
            pltpu.SemaphoreType.DMA,
            pltpu.SemaphoreType.DMA,
            pltpu.SemaphoreType.DMA,
        ],
        compiler_params=pltpu.CompilerParams(needs_layout_passes=False, use_tc_tiling_on_sc=False),
    )
    def edge_kernel(src_hbm, dst_hbm, ew_hbm, prep_hbm, zeros_hbm,
                    out0, out1, src_v, dst_v, ew_v, rows_v, agg_sh,
                    isem0, isem1, isem2, gsem0, gsem1, ssem0, ssem1):
        cid = lax.axis_index("c")
        sid = lax.axis_index("s")
        wid = sid * n_cores + cid
        isem = (isem0, isem1, isem2)
        gsem = (gsem0, gsem1)
        ssem = (ssem0, ssem1)

        base = pl.multiple_of(sid * rpt, 8)

        @pl.when(sid < n_sub - 1)
        def _():
            pltpu.sync_copy(zeros_hbm.at[pl.ds(base, rpt)],
                            agg_sh.at[pl.ds(base, rpt)])

        @pl.when(sid == n_sub - 1)
        def _():
            pltpu.sync_copy(zeros_hbm.at[pl.ds(rpt * (n_sub - 1), rpt_last)],
                            agg_sh.at[pl.ds(rpt * (n_sub - 1), rpt_last)])

        plsc.subcore_barrier()

        def e0_of(i):
            return pl.multiple_of((wid + i * n_workers) * _CHUNK, _CHUNK)

        def idx_descs(i, ib):
            e0 = e0_of(i)
            d = []
            for j in range(_NSUB):
                d.append(pltpu.make_async_copy(
                    src_hbm.at[pl.ds(e0 + j * _SUB, _SUB)],
                    src_v.at[ib].at[j], isem[ib]))
                d.append(pltpu.make_async_copy(
                    dst_hbm.at[pl.ds(e0 + j * _SUB, _SUB)],
                    dst_v.at[ib].at[j], isem[ib]))
            d.append(pltpu.make_async_copy(
                ew_hbm.at[pl.ds(e0, _CHUNK)], ew_v.at[ib], isem[ib]))
            return d

        def gather_descs(i, rb, ib):
            return [pltpu.make_async_copy(
                        prep_hbm.at[src_v.at[ib].at[j]],
                        rows_v.at[rb].at[pl.ds(j * _SUB, _SUB)], gsem[rb])
                    for j in range(_NSUB)]

        def scatter_descs(i, rb, ib):
            return [pltpu.make_async_copy(
                        rows_v.at[rb].at[pl.ds(j * _SUB, _SUB)],
                        agg_sh.at[dst_v.at[ib].at[j]], ssem[rb])
                    for j in range(_NSUB)]

        def start(descs, **kw):
            for d in descs:
                d.start(**kw)

        def drain(descs):
            for d in descs:
                d.wait()

        def scale(rb, ib):
            def body(e, _):
                s = plsc.load_gather(
                    ew_v, [jnp.full((16,), ib, jnp.int32),
                           jnp.full((16,), 0, jnp.int32) + e])
                for k in range(h // 16):
                    col = pl.ds(k * 16, 16)
                    rows_v[rb, e, col] = rows_v[rb, e, col] * s
                return _

            lax.fori_loop(0, _CHUNK, body, 0, unroll=8)

        start(idx_descs(0, 0))
        start(idx_descs(1, 1))
        drain(idx_descs(0, 0))
        start(gather_descs(0, 0, 0))
        for i in range(_NCH):
            rb = i % 2
            ib = i % 3
            drain(gather_descs(i, rb, ib))
            if i >= 1:
                drain(scatter_descs(i - 1, 1 - rb, (i - 1) % 3))
            if i + 1 < _NCH:
                drain(idx_descs(i + 1, (i + 1) % 3))
                start(gather_descs(i + 1, 1 - rb, (i + 1) % 3))
            if i + 2 < _NCH:
                start(idx_descs(i + 2, (i + 2) % 3))
            scale(rb, ib)
            start(scatter_descs(i, rb, ib), add=True)
        drain(scatter_descs(_NCH - 1, (_NCH - 1) % 2, (_NCH - 1) % 3))

        plsc.subcore_barrier()
        sl = pl.ds(base, rpt)
        sl_last = pl.ds(rpt * (n_sub - 1), rpt_last)

        @pl.when((cid == 0) & (sid < n_sub - 1))
        def _():
            pltpu.sync_copy(agg_sh.at[sl], out0.at[sl])

        @pl.when((cid == 0) & (sid == n_sub - 1))
        def _():
            pltpu.sync_copy(agg_sh.at[sl_last], out0.at[sl_last])

        @pl.when((cid == 1) & (sid < n_sub - 1))
        def _():
            pltpu.sync_copy(agg_sh.at[sl], out1.at[sl])

        @pl.when((cid == 1) & (sid == n_sub - 1))
        def _():
            pltpu.sync_copy(agg_sh.at[sl_last], out1.at[sl_last])

    return edge_kernel


def _tc2_body(x, a0, a1, s, wu0x, wu0a, bu0, wu1, bu1, wq0, bq0, wq1, bq1,
              wl, bl, out):
    xv = x[...]
    agg = (a0[...] + a1[...]) * (1.0 / s[...])
    u = _gelu(jnp.dot(xv, wu0x[...], preferred_element_type=jnp.float32)
              + jnp.dot(agg, wu0a[...], preferred_element_type=jnp.float32)
              + bu0[...])
    u = _gelu(jnp.dot(u, wu1[...], preferred_element_type=jnp.float32) + bu1[...])
    u = u * lax.rsqrt(jnp.maximum(jnp.sum(u * u, axis=-1, keepdims=True), 1e-12))
    xv = u + xv
    q = _gelu(jnp.dot(xv, wq0[...], preferred_element_type=jnp.float32) + bq0[...])
    q = _gelu(jnp.dot(q, wq1[...], preferred_element_type=jnp.float32) + bq1[...])
    out[...] = jnp.dot(q, wl[...], preferred_element_type=jnp.float32) + bl[...]


def _sc_gather_kernel(n_rows, d, b):
    n_cores, n_sub = _SC_CORES, _SC_SUBCORES
    mesh = plsc.VectorSubcoreMesh(core_axis_name="c", subcore_axis_name="s",
                                  num_cores=n_cores, num_subcores=n_sub)
    n_workers = n_cores * n_sub
    b_per_w = b // n_workers

    @functools.partial(
        pl.kernel,
        mesh=mesh,
        out_type=jax.ShapeDtypeStruct((b, d), jnp.float32),
        scratch_types=[
            pltpu.VMEM((b_per_w,), jnp.int32),
            pltpu.VMEM((b_per_w, d), jnp.float32),
            pltpu.SemaphoreType.DMA,
        ],
        compiler_params=pltpu.CompilerParams(needs_layout_passes=False, use_tc_tiling_on_sc=False),
    )
    def gather_kernel(table_hbm, idx_hbm, out_hbm, idx_v, rows_v, sem):
        wid = lax.axis_index("s") * n_cores + lax.axis_index("c")
        base = pl.multiple_of(wid * b_per_w, 8)
        pltpu.sync_copy(idx_hbm.at[pl.ds(base, b_per_w)], idx_v)
        pltpu.async_copy(table_hbm.at[idx_v], rows_v, sem).wait()
        pltpu.sync_copy(rows_v, out_hbm.at[pl.ds(base, b_per_w)])

    return gather_kernel


def kernel(node_features, edges, edge_weights, input_node_indices,
           W_pre0, b_pre0, W_pre1, b_pre1,
           W_prep0, b_prep0, W_prep1, b_prep1,
           W_upd0, b_upd0, W_upd1, b_upd1,
           W_post0, b_post0, W_post1, b_post1,
           W_log, b_log):
    n, df = node_features.shape
    e = edge_weights.shape[0]
    h = W_pre0.shape[1]
    nc = W_log.shape[1]
    b = input_node_indices.shape[0]

    ew2d = edge_weights.reshape(e // 128, 128)

    x, prep, s = pl.pallas_call(
        _tc1_body,
        out_shape=[
            jax.ShapeDtypeStruct((n, h), jnp.float32),
            jax.ShapeDtypeStruct((n, h), jnp.float32),
            jax.ShapeDtypeStruct((1, 1), jnp.float32),
        ],
    )(node_features, ew2d,
      W_pre0, b_pre0.reshape(1, h), W_pre1, b_pre1.reshape(1, h),
      W_prep0, b_prep0.reshape(1, h), W_prep1, b_prep1.reshape(1, h))

    e_pad = _SC_CORES * _SC_SUBCORES * _NCH * _CHUNK
    pad = e_pad - e
    src = jnp.pad(edges[1].astype(jnp.int32), (0, pad))
    dst = jnp.pad(edges[0].astype(jnp.int32), (0, pad))
    ewp = jnp.pad(edge_weights, (0, pad))
    zeros = jnp.zeros((n, h), jnp.float32)
    agg0, agg1 = _sc_edge_kernel(n, e_pad, h)(src, dst, ewp, prep, zeros)

    logits_all = pl.pallas_call(
        _tc2_body,
        out_shape=jax.ShapeDtypeStruct((n, nc), jnp.float32),
    )(x, agg0, agg1, s,
      W_upd0[:h], W_upd0[h:], b_upd0.reshape(1, h),
      W_upd1, b_upd1.reshape(1, h),
      W_post0, b_post0.reshape(1, h), W_post1, b_post1.reshape(1, h),
      W_log, b_log.reshape(1, nc))

    idx = input_node_indices.astype(jnp.int32)
    return _sc_gather_kernel(n, nc, b)(logits_all, idx)


# speedup vs baseline: 1.4108x; 1.4108x over previous
import functools
import math

import jax
import jax.numpy as jnp
from jax import lax
from jax.experimental import pallas as pl
from jax.experimental.pallas import tpu as pltpu
from jax.experimental.pallas import tpu_sc as plsc

_SQRT2 = math.sqrt(2.0)


def _gelu(v):
    return 0.5 * v * (1.0 + lax.erf(v / _SQRT2))


def _tc1_body(nf, ew, w0, b0, w1, b1, wp0, bp0, wp1, bp1, x_out, prep_out, s_out):
    x = _gelu(jnp.dot(nf[...], w0[...], preferred_element_type=jnp.float32) + b0[...])
    x = _gelu(jnp.dot(x, w1[...], preferred_element_type=jnp.float32) + b1[...])
    x_out[...] = x
    p = _gelu(jnp.dot(x, wp0[...], preferred_element_type=jnp.float32) + bp0[...])
    p = _gelu(jnp.dot(p, wp1[...], preferred_element_type=jnp.float32) + bp1[...])
    prep_out[...] = p
    s_out[...] = jnp.sum(ew[...], keepdims=True).reshape(1, 1)


_SC_CORES = 2
_SC_SUBCORES = 16
_SUB = 128
_CHUNK = 512
_NSUB = _CHUNK // _SUB


def _sc_edge_kernel(n_nodes, n_edges, h):
    n_chunks = n_edges // _CHUNK
    n_cores, n_sub = _SC_CORES, _SC_SUBCORES
    mesh = plsc.VectorSubcoreMesh(core_axis_name="c", subcore_axis_name="s",
                                  num_cores=n_cores, num_subcores=n_sub)
    n_workers = n_cores * n_sub
    rpt = (-(-n_nodes // n_sub) + 7) // 8 * 8
    rpt_last = n_nodes - rpt * (n_sub - 1)
    assert rpt % 8 == 0 and rpt_last > 0

    @functools.partial(
        pl.kernel,
        mesh=mesh,
        out_type=[
            jax.ShapeDtypeStruct((n_nodes, h), jnp.float32),
            jax.ShapeDtypeStruct((n_nodes, h), jnp.float32),
        ],
        scratch_types=[
            pltpu.VMEM((_NSUB, _SUB), jnp.int32),
            pltpu.VMEM((_NSUB, _SUB), jnp.int32),
            pltpu.VMEM((_CHUNK,), jnp.float32),
            pltpu.VMEM((_CHUNK, h), jnp.float32),
            pltpu.VMEM_SHARED((n_nodes, h), jnp.float32),
            pltpu.SemaphoreType.DMA,
            pltpu.SemaphoreType.DMA,
            pltpu.SemaphoreType.DMA,
        ],
        compiler_params=pltpu.CompilerParams(needs_layout_passes=False, use_tc_tiling_on_sc=False),
    )
    def edge_kernel(src_hbm, dst_hbm, ew_hbm, prep_hbm, zeros_hbm,
                    out0, out1, src_v, dst_v, ew_v, rows_v, agg_sh, sem,
                    isem, ssem):
        cid = lax.axis_index("c")
        sid = lax.axis_index("s")
        wid = sid * n_cores + cid

        base = pl.multiple_of(sid * rpt, 8)

        @pl.when(sid < n_sub - 1)
        def _():
            pltpu.sync_copy(zeros_hbm.at[pl.ds(base, rpt)],
                            agg_sh.at[pl.ds(base, rpt)])

        @pl.when(sid == n_sub - 1)
        def _():
            pltpu.sync_copy(zeros_hbm.at[pl.ds(rpt * (n_sub - 1), rpt_last)],
                            agg_sh.at[pl.ds(rpt * (n_sub - 1), rpt_last)])

        plsc.subcore_barrier()

        def do_chunk(i, _):
            chunk = wid + i * n_workers
            r0 = pl.multiple_of(chunk * _NSUB, _NSUB)
            e0 = pl.multiple_of(chunk * _CHUNK, _CHUNK)
            idescs = [
                pltpu.make_async_copy(src_hbm.at[pl.ds(r0, _NSUB)], src_v, isem),
                pltpu.make_async_copy(dst_hbm.at[pl.ds(r0, _NSUB)], dst_v, isem),
                pltpu.make_async_copy(ew_hbm.at[pl.ds(e0, _CHUNK)], ew_v, isem),
            ]
            for d in idescs:
                d.start()
            for d in idescs:
                d.wait()
            descs = [pltpu.async_copy(prep_hbm.at[src_v.at[j]],
                                      rows_v.at[pl.ds(j * _SUB, _SUB)], sem)
                     for j in range(_NSUB)]
            for d in descs:
                d.wait()

            def scale(e, _):
                s = plsc.load_gather(ew_v, [jnp.full((16,), e, jnp.int32)])
                for c in range(h // 16):
                    col = pl.ds(c * 16, 16)
                    rows_v[e, col] = rows_v[e, col] * s
                return _

            lax.fori_loop(0, _CHUNK, scale, 0, unroll=8)

            sdescs = [pltpu.make_async_copy(rows_v.at[pl.ds(j * _SUB, _SUB)],
                                            agg_sh.at[dst_v.at[j]], ssem)
                      for j in range(_NSUB)]
            for d in sdescs:
                d.start(add=True)
            for d in sdescs:
                d.wait()
            return _

        n_mine = n_chunks // n_workers + jnp.where(wid < n_chunks % n_workers, 1, 0)
        lax.fori_loop(0, n_mine, do_chunk, 0)

        plsc.subcore_barrier()
        sl = pl.ds(base, rpt)
        sl_last = pl.ds(rpt * (n_sub - 1), rpt_last)

        @pl.when((cid == 0) & (sid < n_sub - 1))
        def _():
            pltpu.sync_copy(agg_sh.at[sl], out0.at[sl])

        @pl.when((cid == 0) & (sid == n_sub - 1))
        def _():
            pltpu.sync_copy(agg_sh.at[sl_last], out0.at[sl_last])

        @pl.when((cid == 1) & (sid < n_sub - 1))
        def _():
            pltpu.sync_copy(agg_sh.at[sl], out1.at[sl])

        @pl.when((cid == 1) & (sid == n_sub - 1))
        def _():
            pltpu.sync_copy(agg_sh.at[sl_last], out1.at[sl_last])

    return edge_kernel


def _tc2_body(x, a0, a1, s, wu0x, wu0a, bu0, wu1, bu1, wq0, bq0, wq1, bq1,
              wl, bl, out):
    xv = x[...]
    agg = (a0[...] + a1[...]) * (1.0 / s[...])
    u = _gelu(jnp.dot(xv, wu0x[...], preferred_element_type=jnp.float32)
              + jnp.dot(agg, wu0a[...], preferred_element_type=jnp.float32)
              + bu0[...])
    u = _gelu(jnp.dot(u, wu1[...], preferred_element_type=jnp.float32) + bu1[...])
    u = u * lax.rsqrt(jnp.maximum(jnp.sum(u * u, axis=-1, keepdims=True), 1e-12))
    xv = u + xv
    q = _gelu(jnp.dot(xv, wq0[...], preferred_element_type=jnp.float32) + bq0[...])
    q = _gelu(jnp.dot(q, wq1[...], preferred_element_type=jnp.float32) + bq1[...])
    out[...] = jnp.dot(q, wl[...], preferred_element_type=jnp.float32) + bl[...]


def _sc_gather_kernel(n_rows, d, b):
    n_cores, n_sub = _SC_CORES, _SC_SUBCORES
    mesh = plsc.VectorSubcoreMesh(core_axis_name="c", subcore_axis_name="s",
                                  num_cores=n_cores, num_subcores=n_sub)
    n_workers = n_cores * n_sub
    b_per_w = b // n_workers

    @functools.partial(
        pl.kernel,
        mesh=mesh,
        out_type=jax.ShapeDtypeStruct((b, d), jnp.float32),
        scratch_types=[
            pltpu.VMEM((b_per_w,), jnp.int32),
            pltpu.VMEM((b_per_w, d), jnp.float32),
            pltpu.SemaphoreType.DMA,
        ],
        compiler_params=pltpu.CompilerParams(needs_layout_passes=False, use_tc_tiling_on_sc=False),
    )
    def gather_kernel(table_hbm, idx_hbm, out_hbm, idx_v, rows_v, sem):
        wid = lax.axis_index("s") * n_cores + lax.axis_index("c")
        base = pl.multiple_of(wid * b_per_w, 8)
        pltpu.sync_copy(idx_hbm.at[pl.ds(base, b_per_w)], idx_v)
        pltpu.async_copy(table_hbm.at[idx_v], rows_v, sem).wait()
        pltpu.sync_copy(rows_v, out_hbm.at[pl.ds(base, b_per_w)])

    return gather_kernel


def kernel(node_features, edges, edge_weights, input_node_indices,
           W_pre0, b_pre0, W_pre1, b_pre1,
           W_prep0, b_prep0, W_prep1, b_prep1,
           W_upd0, b_upd0, W_upd1, b_upd1,
           W_post0, b_post0, W_post1, b_post1,
           W_log, b_log):
    n, df = node_features.shape
    e = edge_weights.shape[0]
    h = W_pre0.shape[1]
    nc = W_log.shape[1]
    b = input_node_indices.shape[0]

    ew2d = edge_weights.reshape(e // 128, 128)

    x, prep, s = pl.pallas_call(
        _tc1_body,
        out_shape=[
            jax.ShapeDtypeStruct((n, h), jnp.float32),
            jax.ShapeDtypeStruct((n, h), jnp.float32),
            jax.ShapeDtypeStruct((1, 1), jnp.float32),
        ],
    )(node_features, ew2d,
      W_pre0, b_pre0.reshape(1, h), W_pre1, b_pre1.reshape(1, h),
      W_prep0, b_prep0.reshape(1, h), W_prep1, b_prep1.reshape(1, h))

    src = edges[1].astype(jnp.int32).reshape(-1, _SUB)
    dst = edges[0].astype(jnp.int32).reshape(-1, _SUB)
    zeros = jnp.zeros((n, h), jnp.float32)
    agg0, agg1 = _sc_edge_kernel(n, e, h)(src, dst, edge_weights, prep, zeros)

    logits_all = pl.pallas_call(
        _tc2_body,
        out_shape=jax.ShapeDtypeStruct((n, nc), jnp.float32),
    )(x, agg0, agg1, s,
      W_upd0[:h], W_upd0[h:], b_upd0.reshape(1, h),
      W_upd1, b_upd1.reshape(1, h),
      W_post0, b_post0.reshape(1, h), W_post1, b_post1.reshape(1, h),
      W_log, b_log.reshape(1, nc))

    idx = input_node_indices.astype(jnp.int32)
    return _sc_gather_kernel(n, nc, b)(logits_all, idx)


# speedup vs baseline: 1.7255x; 1.2230x over previous
import functools
import math

import jax
import jax.numpy as jnp
from jax import lax
from jax.experimental import pallas as pl
from jax.experimental.pallas import tpu as pltpu
from jax.experimental.pallas import tpu_sc as plsc

_SQRT2 = math.sqrt(2.0)


def _gelu(v):
    return 0.5 * v * (1.0 + lax.erf(v / _SQRT2))


def _tc1_body(nf, ew, w0, b0, w1, b1, wp0, bp0, wp1, bp1, x_out, prep_out, s_out):
    x = _gelu(jnp.dot(nf[...], w0[...], preferred_element_type=jnp.float32) + b0[...])
    x = _gelu(jnp.dot(x, w1[...], preferred_element_type=jnp.float32) + b1[...])
    x_out[...] = x
    p = _gelu(jnp.dot(x, wp0[...], preferred_element_type=jnp.float32) + bp0[...])
    p = _gelu(jnp.dot(p, wp1[...], preferred_element_type=jnp.float32) + bp1[...])
    prep_out[...] = p
    s_out[...] = jnp.sum(ew[...], keepdims=True).reshape(1, 1)


_SC_CORES = 2
_SC_SUBCORES = 16
_SUB = 128
_CHUNK = 512
_NSUB = _CHUNK // _SUB


def _sc_edge_kernel(n_nodes, n_edges, h):
    n_chunks = n_edges // _CHUNK
    n_cores, n_sub = _SC_CORES, _SC_SUBCORES
    mesh = plsc.VectorSubcoreMesh(core_axis_name="c", subcore_axis_name="s",
                                  num_cores=n_cores, num_subcores=n_sub)
    n_workers = n_cores * n_sub
    rpt = (-(-n_nodes // n_sub) + 7) // 8 * 8
    rpt_last = n_nodes - rpt * (n_sub - 1)
    assert rpt % 8 == 0 and rpt_last > 0

    @functools.partial(
        pl.kernel,
        mesh=mesh,
        out_type=[
            jax.ShapeDtypeStruct((n_nodes, h), jnp.float32),
            jax.ShapeDtypeStruct((n_nodes, h), jnp.float32),
        ],
        scratch_types=[
            pltpu.VMEM((_NSUB, _SUB), jnp.int32),
            pltpu.VMEM((_NSUB, _SUB), jnp.int32),
            pltpu.VMEM((_CHUNK,), jnp.float32),
            pltpu.VMEM((_CHUNK, h), jnp.float32),
            pltpu.VMEM_SHARED((n_nodes, h), jnp.float32),
            pltpu.SemaphoreType.DMA,
            pltpu.SemaphoreType.DMA,
            pltpu.SemaphoreType.DMA,
        ],
        compiler_params=pltpu.CompilerParams(needs_layout_passes=False, use_tc_tiling_on_sc=False),
    )
    def edge_kernel(src_hbm, dst_hbm, ew_hbm, prep_hbm, zeros_hbm,
                    out0, out1, src_v, dst_v, ew_v, rows_v, agg_sh, sem,
                    isem, ssem):
        cid = lax.axis_index("c")
        sid = lax.axis_index("s")
        wid = sid * n_cores + cid

        base = pl.multiple_of(sid * rpt, 8)

        @pl.when(sid < n_sub - 1)
        def _():
            pltpu.sync_copy(zeros_hbm.at[pl.ds(base, rpt)],
                            agg_sh.at[pl.ds(base, rpt)])

        @pl.when(sid == n_sub - 1)
        def _():
            pltpu.sync_copy(zeros_hbm.at[pl.ds(rpt * (n_sub - 1), rpt_last)],
                            agg_sh.at[pl.ds(rpt * (n_sub - 1), rpt_last)])

        plsc.subcore_barrier()

        def do_chunk(i, _):
            chunk = wid + i * n_workers
            r0 = pl.multiple_of(chunk * _NSUB, _NSUB)
            e0 = pl.multiple_of(chunk * _CHUNK, _CHUNK)
            idescs = [
                pltpu.make_async_copy(src_hbm.at[pl.ds(r0, _NSUB)], src_v, isem),
                pltpu.make_async_copy(dst_hbm.at[pl.ds(r0, _NSUB)], dst_v, isem),
                pltpu.make_async_copy(ew_hbm.at[pl.ds(e0, _CHUNK)], ew_v, isem),
            ]
            for d in idescs:
                d.start()
            for d in idescs:
                d.wait()
            descs = [pltpu.async_copy(prep_hbm.at[src_v.at[j]],
                                      rows_v.at[pl.ds(j * _SUB, _SUB)], sem)
                     for j in range(_NSUB)]
            for d in descs:
                d.wait()

            @plsc.parallel_loop(0, _CHUNK, unroll=8)
            def _scale(e):
                s = plsc.load_gather(ew_v, [jnp.full((16,), e, jnp.int32)])
                for c in range(h // 16):
                    col = pl.ds(c * 16, 16)
                    rows_v[e, col] = rows_v[e, col] * s

            sdescs = [pltpu.make_async_copy(rows_v.at[pl.ds(j * _SUB, _SUB)],
                                            agg_sh.at[dst_v.at[j]], ssem)
                      for j in range(_NSUB)]
            for d in sdescs:
                d.start(add=True)
            for d in sdescs:
                d.wait()
            return _

        n_mine = n_chunks // n_workers + jnp.where(wid < n_chunks % n_workers, 1, 0)
        lax.fori_loop(0, n_mine, do_chunk, 0)

        plsc.subcore_barrier()
        sl = pl.ds(base, rpt)
        sl_last = pl.ds(rpt * (n_sub - 1), rpt_last)

        @pl.when((cid == 0) & (sid < n_sub - 1))
        def _():
            pltpu.sync_copy(agg_sh.at[sl], out0.at[sl])

        @pl.when((cid == 0) & (sid == n_sub - 1))
        def _():
            pltpu.sync_copy(agg_sh.at[sl_last], out0.at[sl_last])

        @pl.when((cid == 1) & (sid < n_sub - 1))
        def _():
            pltpu.sync_copy(agg_sh.at[sl], out1.at[sl])

        @pl.when((cid == 1) & (sid == n_sub - 1))
        def _():
            pltpu.sync_copy(agg_sh.at[sl_last], out1.at[sl_last])

    return edge_kernel


def _tc2_body(x, a0, a1, s, wu0x, wu0a, bu0, wu1, bu1, wq0, bq0, wq1, bq1,
              wl, bl, out):
    xv = x[...]
    agg = (a0[...] + a1[...]) * (1.0 / s[...])
    u = _gelu(jnp.dot(xv, wu0x[...], preferred_element_type=jnp.float32)
              + jnp.dot(agg, wu0a[...], preferred_element_type=jnp.float32)
              + bu0[...])
    u = _gelu(jnp.dot(u, wu1[...], preferred_element_type=jnp.float32) + bu1[...])
    u = u * lax.rsqrt(jnp.maximum(jnp.sum(u * u, axis=-1, keepdims=True), 1e-12))
    xv = u + xv
    q = _gelu(jnp.dot(xv, wq0[...], preferred_element_type=jnp.float32) + bq0[...])
    q = _gelu(jnp.dot(q, wq1[...], preferred_element_type=jnp.float32) + bq1[...])
    out[...] = jnp.dot(q, wl[...], preferred_element_type=jnp.float32) + bl[...]


def _sc_gather_kernel(n_rows, d, b):
    n_cores, n_sub = _SC_CORES, _SC_SUBCORES
    mesh = plsc.VectorSubcoreMesh(core_axis_name="c", subcore_axis_name="s",
                                  num_cores=n_cores, num_subcores=n_sub)
    n_workers = n_cores * n_sub
    b_per_w = b // n_workers

    @functools.partial(
        pl.kernel,
        mesh=mesh,
        out_type=jax.ShapeDtypeStruct((b, d), jnp.float32),
        scratch_types=[
            pltpu.VMEM((b_per_w,), jnp.int32),
            pltpu.VMEM((b_per_w, d), jnp.float32),
            pltpu.SemaphoreType.DMA,
        ],
        compiler_params=pltpu.CompilerParams(needs_layout_passes=False, use_tc_tiling_on_sc=False),
    )
    def gather_kernel(table_hbm, idx_hbm, out_hbm, idx_v, rows_v, sem):
        wid = lax.axis_index("s") * n_cores + lax.axis_index("c")
        base = pl.multiple_of(wid * b_per_w, 8)
        pltpu.sync_copy(idx_hbm.at[pl.ds(base, b_per_w)], idx_v)
        pltpu.async_copy(table_hbm.at[idx_v], rows_v, sem).wait()
        pltpu.sync_copy(rows_v, out_hbm.at[pl.ds(base, b_per_w)])

    return gather_kernel


def kernel(node_features, edges, edge_weights, input_node_indices,
           W_pre0, b_pre0, W_pre1, b_pre1,
           W_prep0, b_prep0, W_prep1, b_prep1,
           W_upd0, b_upd0, W_upd1, b_upd1,
           W_post0, b_post0, W_post1, b_post1,
           W_log, b_log):
    n, df = node_features.shape
    e = edge_weights.shape[0]
    h = W_pre0.shape[1]
    nc = W_log.shape[1]
    b = input_node_indices.shape[0]

    ew2d = edge_weights.reshape(e // 128, 128)

    x, prep, s = pl.pallas_call(
        _tc1_body,
        out_shape=[
            jax.ShapeDtypeStruct((n, h), jnp.float32),
            jax.ShapeDtypeStruct((n, h), jnp.float32),
            jax.ShapeDtypeStruct((1, 1), jnp.float32),
        ],
    )(node_features, ew2d,
      W_pre0, b_pre0.reshape(1, h), W_pre1, b_pre1.reshape(1, h),
      W_prep0, b_prep0.reshape(1, h), W_prep1, b_prep1.reshape(1, h))

    src = edges[1].astype(jnp.int32).reshape(-1, _SUB)
    dst = edges[0].astype(jnp.int32).reshape(-1, _SUB)
    zeros = jnp.zeros((n, h), jnp.float32)
    agg0, agg1 = _sc_edge_kernel(n, e, h)(src, dst, edge_weights, prep, zeros)

    logits_all = pl.pallas_call(
        _tc2_body,
        out_shape=jax.ShapeDtypeStruct((n, nc), jnp.float32),
    )(x, agg0, agg1, s,
      W_upd0[:h], W_upd0[h:], b_upd0.reshape(1, h),
      W_upd1, b_upd1.reshape(1, h),
      W_post0, b_post0.reshape(1, h), W_post1, b_post1.reshape(1, h),
      W_log, b_log.reshape(1, nc))

    idx = input_node_indices.astype(jnp.int32)
    return _sc_gather_kernel(n, nc, b)(logits_all, idx)
